# baseline (speedup 1.0000x reference)
import functools

import jax
import jax.numpy as jnp
from jax import lax
from jax.experimental import pallas as pl
from jax.experimental.pallas import tpu as pltpu
from jax.experimental.pallas import tpu_sc as plsc

B = 16384
DIM = 64
RATE_REG = 0.0001
LANES = 16
NC, NS = 2, 16
NW = NC * NS
BPW = B // NW
GCHUNK = 128
NCH = BPW // GCHUNK
PAIR = 2 * DIM

_MESH = dict(core_axis_name="c", subcore_axis_name="s")


def _wid_base():
    wid = lax.axis_index("s") * NC + lax.axis_index("c")
    return wid, wid * BPW


def _user_kernel(hu_hbm, ou_hbm, eu_hbm,
                 sel_out, sq_out,
                 hu_v, ou_v, b0, b1, sel_v, sq_acc, sems):
    wid, base = _wid_base()
    pltpu.sync_copy(hu_hbm.at[pl.ds(base, BPW)], hu_v)
    pltpu.sync_copy(ou_hbm.at[pl.ds(base, BPW)], ou_v)

    bufs = (b0, b1)

    def start(j):
        sl = pl.ds(j * GCHUNK, GCHUNK)
        return pltpu.async_copy(eu_hbm.at[hu_v.at[sl]], bufs[j % 2],
                                sems.at[j % 2])

    sq_acc[...] = jnp.zeros((LANES,), jnp.float32)

    pending = {0: start(0)}
    for j in range(NCH):
        if j + 1 < NCH:
            pending[j + 1] = start(j + 1)
        pending.pop(j).wait()
        bu = bufs[j % 2]
        row0 = j * GCHUNK

        @pl.loop(0, GCHUNK // LANES)
        def _(g, bu=bu, row0=row0):
            r0 = row0 + g * LANES
            ouv = ou_v[pl.ds(r0, LANES)]
            for l in range(LANES):
                pu = ouv[l]
                i = g * LANES + l
                s_vec = None
                for c in range(DIM // LANES):
                    u = bu[i, pl.ds(pu + c * LANES, LANES)]
                    sel_v[r0 + l, pl.ds(c * LANES, LANES)] = u
                    s_c = u * u
                    s_vec = s_c if s_vec is None else s_vec + s_c
                sq_acc[...] = sq_acc[...] + s_vec

    pltpu.sync_copy(sel_v, sel_out.at[pl.ds(base, BPW)])
    pltpu.sync_copy(sq_acc, sq_out.at[wid])


def _item_kernel(hp_hbm, hn_hbm, op_hbm, on_hbm, eu_sel_hbm, ei_hbm,
                 d_out, sq_out,
                 hp_v, hn_v, op_v, on_v, euv,
                 bp0, bp1, bn0, bn1, d_part, sq_acc, sems):
    wid, base = _wid_base()
    pltpu.sync_copy(hp_hbm.at[pl.ds(base, BPW)], hp_v)
    pltpu.sync_copy(hn_hbm.at[pl.ds(base, BPW)], hn_v)
    pltpu.sync_copy(op_hbm.at[pl.ds(base, BPW)], op_v)
    pltpu.sync_copy(on_hbm.at[pl.ds(base, BPW)], on_v)
    pltpu.sync_copy(eu_sel_hbm.at[pl.ds(base, BPW)], euv)

    bufs = ((bp0, bn0), (bp1, bn1))

    def start(j):
        sl = pl.ds(j * GCHUNK, GCHUNK)
        bp, bn = bufs[j % 2]
        s = j % 2
        return (
            pltpu.async_copy(ei_hbm.at[hp_v.at[sl]], bp, sems.at[s, 0]),
            pltpu.async_copy(ei_hbm.at[hn_v.at[sl]], bn, sems.at[s, 1]),
        )

    sq_acc[...] = jnp.zeros((LANES,), jnp.float32)

    pending = {0: start(0)}
    for j in range(NCH):
        if j + 1 < NCH:
            pending[j + 1] = start(j + 1)
        for c in pending.pop(j):
            c.wait()
        bp, bn = bufs[j % 2]
        row0 = j * GCHUNK

        @pl.loop(0, GCHUNK // LANES)
        def _(g, bp=bp, bn=bn, row0=row0):
            r0 = row0 + g * LANES
            opv = op_v[pl.ds(r0, LANES)]
            onv = on_v[pl.ds(r0, LANES)]
            for l in range(LANES):
                pp = opv[l]
                pn = onv[l]
                i = g * LANES + l
                d_vec = None
                s_vec = None
                for c in range(DIM // LANES):
                    u = euv[r0 + l, pl.ds(c * LANES, LANES)]
                    p = bp[i, pl.ds(pp + c * LANES, LANES)]
                    n = bn[i, pl.ds(pn + c * LANES, LANES)]
                    d_c = u * (p - n)
                    s_c = p * p + n * n
                    d_vec = d_c if d_vec is None else d_vec + d_c
                    s_vec = s_c if s_vec is None else s_vec + s_c
                d_part[r0 + l, :] = d_vec
                sq_acc[...] = sq_acc[...] + s_vec

    pltpu.sync_copy(d_part, d_out.at[pl.ds(base, BPW)])
    pltpu.sync_copy(sq_acc, sq_out.at[wid])


def _sc_user_gather(h_u, o_u, eu2):
    kern = functools.partial(
        pl.kernel,
        mesh=plsc.VectorSubcoreMesh(**_MESH),
        compiler_params=pltpu.CompilerParams(use_tc_tiling_on_sc=False),
        out_type=(
            jax.ShapeDtypeStruct((B, DIM), jnp.float32),
            jax.ShapeDtypeStruct((NW, LANES), jnp.float32),
        ),
        scratch_types=[
            pltpu.VMEM((BPW,), jnp.int32),
            pltpu.VMEM((BPW,), jnp.int32),
            pltpu.VMEM((GCHUNK, PAIR), jnp.float32),
            pltpu.VMEM((GCHUNK, PAIR), jnp.float32),
            pltpu.VMEM((BPW, DIM), jnp.float32),
            pltpu.VMEM((LANES,), jnp.float32),
            pltpu.SemaphoreType.DMA((2,)),
        ],
    )(_user_kernel)
    return kern(h_u, o_u, eu2)


def _sc_item_dot(h_p, h_n, o_p, o_n, eu_sel, ei2):
    kern = functools.partial(
        pl.kernel,
        mesh=plsc.VectorSubcoreMesh(**_MESH),
        compiler_params=pltpu.CompilerParams(use_tc_tiling_on_sc=False),
        out_type=(
            jax.ShapeDtypeStruct((B, LANES), jnp.float32),
            jax.ShapeDtypeStruct((NW, LANES), jnp.float32),
        ),
        scratch_types=[
            pltpu.VMEM((BPW,), jnp.int32),
            pltpu.VMEM((BPW,), jnp.int32),
            pltpu.VMEM((BPW,), jnp.int32),
            pltpu.VMEM((BPW,), jnp.int32),
            pltpu.VMEM((BPW, DIM), jnp.float32),
            pltpu.VMEM((GCHUNK, PAIR), jnp.float32),
            pltpu.VMEM((GCHUNK, PAIR), jnp.float32),
            pltpu.VMEM((GCHUNK, PAIR), jnp.float32),
            pltpu.VMEM((GCHUNK, PAIR), jnp.float32),
            pltpu.VMEM((BPW, LANES), jnp.float32),
            pltpu.VMEM((LANES,), jnp.float32),
            pltpu.SemaphoreType.DMA((2, 2)),
        ],
    )(_item_kernel)
    return kern(h_p, h_n, o_p, o_n, eu_sel, ei2)


def _finish_body(d_ref, squ_ref, sqi_ref, base_ref, reg_ref):
    d = jnp.sum(d_ref[...], axis=1)
    sp = jnp.maximum(-d, 0.0) + jnp.log1p(jnp.exp(-jnp.abs(d)))
    base_ref[0, 0] = jnp.sum(sp) * (1.0 / B)
    reg_ref[0, 0] = (0.5 * RATE_REG) * (jnp.sum(squ_ref[...]) +
                                        jnp.sum(sqi_ref[...]))


def _tc_finish(d_part, squ, sqi):
    return pl.pallas_call(
        _finish_body,
        out_shape=(
            jax.ShapeDtypeStruct((1, 1), jnp.float32),
            jax.ShapeDtypeStruct((1, 1), jnp.float32),
        ),
        out_specs=(
            pl.BlockSpec(memory_space=pltpu.SMEM),
            pl.BlockSpec(memory_space=pltpu.SMEM),
        ),
    )(d_part, squ, sqi)


def kernel(users, items_pos, items_neg, embed_user, embed_item):
    eu2 = embed_user.reshape(embed_user.shape[0] // 2, PAIR)
    ei2 = embed_item.reshape(embed_item.shape[0] // 2, PAIR)
    h_u = lax.shift_right_logical(users, 1)
    h_p = lax.shift_right_logical(items_pos, 1)
    h_n = lax.shift_right_logical(items_neg, 1)
    o_u = (users & 1) * DIM
    o_p = (items_pos & 1) * DIM
    o_n = (items_neg & 1) * DIM
    eu_sel, squ = _sc_user_gather(h_u, o_u, eu2)
    d_part, sqi = _sc_item_dot(h_p, h_n, o_p, o_n, eu_sel, ei2)
    base2d, reg2d = _tc_finish(d_part, squ, sqi)
    return base2d[0, 0], reg2d[0, 0]


# speedup vs baseline: 1.0039x; 1.0039x over previous
import functools

import jax
import jax.numpy as jnp
from jax import lax
from jax.experimental import pallas as pl
from jax.experimental.pallas import tpu as pltpu
from jax.experimental.pallas import tpu_sc as plsc

B = 16384
DIM = 64
RATE_REG = 0.0001
LANES = 16
NC, NS = 2, 16
NW = NC * NS
BPW = B // NW
GCHUNK = 128
NCH = BPW // GCHUNK


def _sc_kernel(users_hbm, pos_hbm, neg_hbm, eu_hbm, ei_hbm,
               d_out, sq_out,
               idx_u, idx_p, idx_n, rows_u, rows_p, rows_n,
               d_part, sq_acc, sems):
    wid = lax.axis_index("s") * NC + lax.axis_index("c")
    base = wid * BPW

    pltpu.sync_copy(users_hbm.at[pl.ds(base, BPW)], idx_u)
    pltpu.sync_copy(pos_hbm.at[pl.ds(base, BPW)], idx_p)
    pltpu.sync_copy(neg_hbm.at[pl.ds(base, BPW)], idx_n)

    copies = []
    for j in range(NCH):
        sl = pl.ds(j * GCHUNK, GCHUNK)
        copies.append(pltpu.async_copy(
            eu_hbm.at[idx_u.at[sl]], rows_u.at[sl], sems.at[0]))
        copies.append(pltpu.async_copy(
            ei_hbm.at[idx_p.at[sl]], rows_p.at[sl], sems.at[1]))
        copies.append(pltpu.async_copy(
            ei_hbm.at[idx_n.at[sl]], rows_n.at[sl], sems.at[2]))
    for c in copies:
        c.wait()

    sq_acc[...] = jnp.zeros((LANES,), jnp.float32)

    @pl.loop(0, BPW)
    def _(i):
        d_vec = None
        s_vec = None
        for c in range(DIM // LANES):
            sl = pl.ds(c * LANES, LANES)
            u = rows_u[i, sl]
            p = rows_p[i, sl]
            n = rows_n[i, sl]
            d_c = u * (p - n)
            s_c = u * u + (p * p + n * n)
            d_vec = d_c if d_vec is None else d_vec + d_c
            s_vec = s_c if s_vec is None else s_vec + s_c
        d_part[i, :] = d_vec
        sq_acc[...] = sq_acc[...] + s_vec

    pltpu.sync_copy(d_part, d_out.at[pl.ds(base, BPW)])
    pltpu.sync_copy(sq_acc, sq_out.at[wid])


def _sc_gather_partials(users, items_pos, items_neg, embed_user, embed_item):
    mesh = plsc.VectorSubcoreMesh(core_axis_name="c", subcore_axis_name="s")
    kern = functools.partial(
        pl.kernel,
        mesh=mesh,
        compiler_params=pltpu.CompilerParams(use_tc_tiling_on_sc=False),
        out_type=(
            jax.ShapeDtypeStruct((B, LANES), jnp.float32),
            jax.ShapeDtypeStruct((NW, LANES), jnp.float32),
        ),
        scratch_types=[
            pltpu.VMEM((BPW,), jnp.int32),
            pltpu.VMEM((BPW,), jnp.int32),
            pltpu.VMEM((BPW,), jnp.int32),
            pltpu.VMEM((BPW, DIM), jnp.float32),
            pltpu.VMEM((BPW, DIM), jnp.float32),
            pltpu.VMEM((BPW, DIM), jnp.float32),
            pltpu.VMEM((BPW, LANES), jnp.float32),
            pltpu.VMEM((LANES,), jnp.float32),
            pltpu.SemaphoreType.DMA((3,)),
        ],
    )(_sc_kernel)
    return kern(users, items_pos, items_neg, embed_user, embed_item)


def _finish_body(d_ref, sq_ref, base_ref, reg_ref):
    d = jnp.sum(d_ref[...], axis=1)
    sp = jnp.maximum(-d, 0.0) + jnp.log1p(jnp.exp(-jnp.abs(d)))
    base_ref[0, 0] = jnp.sum(sp) * (1.0 / B)
    reg_ref[0, 0] = (0.5 * RATE_REG) * jnp.sum(sq_ref[...])


def _tc_finish(d_part, sq_part):
    return pl.pallas_call(
        _finish_body,
        out_shape=(
            jax.ShapeDtypeStruct((1, 1), jnp.float32),
            jax.ShapeDtypeStruct((1, 1), jnp.float32),
        ),
        out_specs=(
            pl.BlockSpec(memory_space=pltpu.SMEM),
            pl.BlockSpec(memory_space=pltpu.SMEM),
        ),
    )(d_part, sq_part)


def kernel(users, items_pos, items_neg, embed_user, embed_item):
    d_part, sq_part = _sc_gather_partials(
        users, items_pos, items_neg, embed_user, embed_item)
    base2d, reg2d = _tc_finish(d_part, sq_part)
    return base2d[0, 0], reg2d[0, 0]
